# baseline (speedup 1.0000x reference)
import jax
import jax.numpy as jnp
from jax import lax
from jax.experimental import pallas as pl
from jax.experimental.pallas import tpu as pltpu
from jax.experimental.pallas import tpu_sc as plsc

NC, NS, L = 2, 16, 16
NW = NC * NS
BATCH, SEQ, HID = 16384, 200, 64
BLK = 128
NBLK = BATCH // BLK // NW
N_UNITS = NBLK * SEQ
NBUF = 4
K = 2
HT, HS = HID // 8, 8


def _body(xt_hbm, wtok_hbm, wpos_hbm, out_hbm, idx_v, stage_v, tile_v,
          pos_v, gsem, osem):
    c = lax.axis_index("c")
    sub = lax.axis_index("s")
    wid = sub * NC + c
    pltpu.sync_copy(wpos_hbm.at[pl.ds(0, SEQ)], pos_v)
    blk0 = wid * NBLK
    lane = lax.iota(jnp.int32, 16)

    def unit_sb(it):
        return it & (NBLK - 1), lax.shift_right_logical(it, 2)

    def fire_gathers(it, b):
        blk, s = unit_sb(it)
        bt = blk0 + blk
        pltpu.sync_copy(xt_hbm.at[s, pl.ds(bt * BLK, BLK)], idx_v.at[b])
        pltpu.async_copy(wtok_hbm.at[idx_v.at[b]], stage_v.at[b], gsem.at[b])

    def wait_gather(b):
        pltpu.make_async_copy(wtok_hbm.at[idx_v.at[b]], stage_v.at[b],
                              gsem.at[b]).wait()

    def write_pairs(b, it):
        blk, s = unit_sb(it)
        bt = blk0 + blk
        for ht in range(HT):
            yield tile_v.at[b, ht], out_hbm.at[s, ht, bt]

    def fire_writes(b, it):
        for src, dst in write_pairs(b, it):
            pltpu.async_copy(src, dst, osem.at[b])

    def wait_writes(b, it):
        for src, dst in write_pairs(b, it):
            pltpu.make_async_copy(src, dst, osem.at[b]).wait()

    def assemble(b, it):
        _, s = unit_sb(it)
        s16 = jnp.full((16,), s, jnp.int32)

        def per_ht(ht, acc):
            for hs in range(HS):
                h16 = jnp.full((16,), ht * HS + hs, jnp.int32)
                psplat = plsc.load_gather(pos_v, [s16, h16])
                for g in range(BLK // L):
                    rows = lane + (L * g)
                    vals = plsc.load_gather(stage_v.at[b], [rows, h16])
                    tile_v[b, ht, hs, pl.ds(L * g, L)] = vals + psplat
            return acc

        lax.fori_loop(0, HT, per_ht, 0)

    for b in range(K):
        fire_gathers(b, b)

    def superstep(ss, carry):
        for b in range(NBUF):
            it = ss * NBUF + b
            nxt = it + K
            pb = (b + K) % NBUF
            @pl.when(nxt < N_UNITS)
            def _():
                fire_gathers(nxt, pb)
            wait_gather(b)
            @pl.when(it >= NBUF)
            def _():
                wait_writes(b, it - NBUF)
            assemble(b, it)
            fire_writes(b, it)
        return carry

    lax.fori_loop(0, N_UNITS // NBUF, superstep, 0)

    for it in range(N_UNITS - NBUF, N_UNITS):
        wait_writes(it % NBUF, it)


_mesh = plsc.VectorSubcoreMesh(core_axis_name="c", subcore_axis_name="s")

_gather_add = pl.kernel(
    _body,
    mesh=_mesh,
    compiler_params=pltpu.CompilerParams(use_tc_tiling_on_sc=False,
                                         needs_layout_passes=False),
    out_type=jax.ShapeDtypeStruct((SEQ, HT, BATCH // BLK, HS, BLK),
                                  jnp.float32),
    scratch_types=[
        pltpu.VMEM((NBUF, BLK), jnp.int32),
        pltpu.VMEM((NBUF, BLK, HID), jnp.float32),
        pltpu.VMEM((NBUF, HT, HS, BLK), jnp.float32),
        pltpu.VMEM((SEQ, HID), jnp.float32),
        pltpu.SemaphoreType.DMA((NBUF,)),
        pltpu.SemaphoreType.DMA((NBUF,)),
    ],
)


def kernel(input_X, W_tok, W_pos):
    xt = input_X.astype(jnp.int32).T
    o = _gather_add(xt, W_tok, W_pos)
    return o.transpose(2, 4, 0, 1, 3).reshape(BATCH, SEQ, HID)


# speedup vs baseline: 2.0957x; 2.0957x over previous
import jax
import jax.numpy as jnp
from jax import lax
from jax.experimental import pallas as pl
from jax.experimental.pallas import tpu as pltpu
from jax.experimental.pallas import tpu_sc as plsc

NC, NS, L = 2, 16, 16
NW = NC * NS
BATCH, SEQ, HID = 16384, 200, 64
HPAD = HID
BLK = 128
NBLK = BATCH // BLK // NW
N_UNITS = NBLK * SEQ
NBUF = 4
KI = 4
KG = 2
HT, HS = HID // 8, 8


def _body(xt_hbm, wtok_hbm, wpos_hbm, out_hbm, idx_v, stage_v, tile_v,
          pos_v, isem, gsem, osem):
    c = lax.axis_index("c")
    sub = lax.axis_index("s")
    wid = sub * NC + c
    pltpu.sync_copy(wpos_hbm.at[pl.ds(0, SEQ)], pos_v)
    blk0 = wid * NBLK
    lane = lax.iota(jnp.int32, 16)

    def unit_sb(it):
        return it & (NBLK - 1), lax.shift_right_logical(it, 2)

    def idx_copy(it, b):
        blk, s = unit_sb(it)
        bt = blk0 + blk
        return pltpu.make_async_copy(xt_hbm.at[s, pl.ds(bt * BLK, BLK)],
                                     idx_v.at[b], isem.at[b])

    def gather_copy(it, b):
        del it
        return pltpu.make_async_copy(
            wtok_hbm.at[idx_v.at[b]],
            stage_v.at[b], gsem.at[b])

    def write_pairs(b, it):
        blk, s = unit_sb(it)
        bt = blk0 + blk
        for ht in range(HT):
            yield pltpu.make_async_copy(tile_v.at[b, ht],
                                        out_hbm.at[s, ht, bt], osem.at[b])

    def assemble(b, it):
        _, s = unit_sb(it)
        s16 = jnp.full((16,), s, jnp.int32)

        @plsc.parallel_loop(0, HID, unroll=2)
        def _h(h):
            ht = lax.shift_right_logical(h, 3)
            hs = h & (HS - 1)
            h16 = jnp.full((16,), h, jnp.int32)
            psplat = plsc.load_gather(pos_v, [s16, h16])
            for g in range(BLK // L):
                rows = lane + (L * g)
                vals = plsc.load_gather(stage_v.at[b], [rows, h16])
                tile_v[b, ht, hs, pl.ds(L * g, L)] = vals + psplat

    for b in range(KI):
        idx_copy(b, b).start()
    for b in range(KG):
        idx_copy(b, b).wait()
        gather_copy(b, b).start()

    def superstep(ss, carry):
        for b in range(NBUF):
            it = ss * NBUF + b
            @pl.when(it < N_UNITS)
            def _():
                gather_copy(it, b).wait()
                @pl.when(it + KI < N_UNITS)
                def _():
                    idx_copy(it + KI, b).start()
                pg = (b + KG) % NBUF
                @pl.when(it + KG < N_UNITS)
                def _():
                    idx_copy(it + KG, pg).wait()
                    gather_copy(it + KG, pg).start()
                @pl.when(it >= NBUF)
                def _():
                    for cp in write_pairs(b, it - NBUF):
                        cp.wait()
                assemble(b, it)
                for cp in write_pairs(b, it):
                    cp.start()
        return carry

    lax.fori_loop(0, pl.cdiv(N_UNITS, NBUF), superstep, 0)

    for it in range(N_UNITS - NBUF, N_UNITS):
        for cp in write_pairs(it % NBUF, it):
            cp.wait()


_mesh = plsc.VectorSubcoreMesh(core_axis_name="c", subcore_axis_name="s")

_gather_add = pl.kernel(
    _body,
    mesh=_mesh,
    compiler_params=pltpu.CompilerParams(use_tc_tiling_on_sc=False,
                                         needs_layout_passes=False),
    out_type=jax.ShapeDtypeStruct((SEQ, HT, BATCH // BLK, HS, BLK),
                                  jnp.float32),
    scratch_types=[
        pltpu.VMEM((NBUF, BLK), jnp.int32),
        pltpu.VMEM((NBUF, BLK, HPAD), jnp.float32),
        pltpu.VMEM((NBUF, HT, HS, BLK), jnp.float32),
        pltpu.VMEM((SEQ, HID), jnp.float32),
        pltpu.SemaphoreType.DMA((NBUF,)),
        pltpu.SemaphoreType.DMA((NBUF,)),
        pltpu.SemaphoreType.DMA((NBUF,)),
    ],
)


def kernel(input_X, W_tok, W_pos):
    xt = input_X.astype(jnp.int32).T
    o = _gather_add(xt, W_tok, W_pos)
    return o.transpose(2, 4, 0, 1, 3).reshape(BATCH, SEQ, HID)


# speedup vs baseline: 4.2429x; 2.0246x over previous
import jax
import jax.numpy as jnp
from jax import lax
from jax.experimental import pallas as pl
from jax.experimental.pallas import tpu as pltpu
from jax.experimental.pallas import tpu_sc as plsc

NC, NS, L = 2, 16, 16
NW = NC * NS
BATCH, SEQ, HID = 16384, 200, 64
SEQ_PER_W = BATCH // NW
NSEQ_CHUNK = 2
N_ITERS = SEQ_PER_W // NSEQ_CHUNK
IDX_ROW = 100
G = NSEQ_CHUNK * SEQ // IDX_ROW
NBUF = 4
K = 2


def _body(idx_hbm, wtok_hbm, wpos_hbm, out_hbm, idx_v, rows_v, pos_v,
          gsem, osem):
    c = lax.axis_index("c")
    s = lax.axis_index("s")
    wid = s * NC + c
    pltpu.sync_copy(wpos_hbm.at[pl.ds(0, SEQ)], pos_v)
    seq0 = wid * SEQ_PER_W

    def gather_pairs(b):
        for g in range(G):
            yield (idx_v.at[b, g], rows_v.at[b, pl.ds(g * IDX_ROW, IDX_ROW)])

    def fire_gathers(it, b):
        row0 = (seq0 + it * NSEQ_CHUNK) * (SEQ // IDX_ROW)
        pltpu.sync_copy(idx_hbm.at[pl.ds(row0, G)], idx_v.at[b])
        for isrc, rdst in gather_pairs(b):
            pltpu.async_copy(wtok_hbm.at[isrc], rdst, gsem.at[b])

    def wait_gathers(b):
        for isrc, rdst in gather_pairs(b):
            pltpu.make_async_copy(wtok_hbm.at[isrc], rdst, gsem.at[b]).wait()

    def write_pairs(b, it):
        for n in range(NSEQ_CHUNK):
            yield (rows_v.at[b, pl.ds(n * SEQ, SEQ)],
                   out_hbm.at[seq0 + it * NSEQ_CHUNK + n, pl.ds(0, SEQ),
                              pl.ds(0, HID)])

    def wait_write(b, it):
        for src, dst in write_pairs(b, it):
            pltpu.make_async_copy(src, dst, osem.at[b]).wait()

    def add_pos(b):
        def add_row(j, acc):
            for q in range(HID // L):
                sl = pl.ds(q * L, L)
                p = pos_v[j, sl]
                for n in range(NSEQ_CHUNK):
                    plsc.addupdate(rows_v.at[b, n * SEQ + j, sl], p)
            return acc
        lax.fori_loop(0, SEQ, add_row, 0)

    for b in range(K):
        fire_gathers(b, b)

    def superstep(ss, carry):
        for b in range(NBUF):
            it = ss * NBUF + b
            nxt = it + K
            pb = (b + K) % NBUF
            @pl.when(jnp.logical_and(nxt >= NBUF, nxt < N_ITERS))
            def _():
                wait_write(pb, nxt - NBUF)
            @pl.when(nxt < N_ITERS)
            def _():
                fire_gathers(nxt, pb)
            wait_gathers(b)
            add_pos(b)
            for src_, dst_ in write_pairs(b, it):
                pltpu.async_copy(src_, dst_, osem.at[b])
        return carry

    lax.fori_loop(0, N_ITERS // NBUF, superstep, 0)

    for it in range(N_ITERS - K, N_ITERS):
        wait_write(it % NBUF, it)


_mesh = plsc.VectorSubcoreMesh(core_axis_name="c", subcore_axis_name="s")

_gather_add = pl.kernel(
    _body,
    mesh=_mesh,
    compiler_params=pltpu.CompilerParams(use_tc_tiling_on_sc=False),
    out_type=jax.ShapeDtypeStruct((BATCH, SEQ, 128), jnp.float32),
    scratch_types=[
        pltpu.VMEM((NBUF, G, IDX_ROW), jnp.int32),
        pltpu.VMEM((NBUF, NSEQ_CHUNK * SEQ, HID), jnp.float32),
        pltpu.VMEM((SEQ, HID), jnp.float32),
        pltpu.SemaphoreType.DMA((NBUF,)),
        pltpu.SemaphoreType.DMA((NBUF,)),
    ],
)


def kernel(input_X, W_tok, W_pos):
    idx = input_X.astype(jnp.int32).reshape(BATCH * SEQ // IDX_ROW, IDX_ROW)
    o = _gather_add(idx, W_tok, W_pos)
    return o[:, :, 0:HID]


# speedup vs baseline: 4.2429x; 1.0000x over previous
import jax
import jax.numpy as jnp
from jax import lax
from jax.experimental import pallas as pl
from jax.experimental.pallas import tpu as pltpu
from jax.experimental.pallas import tpu_sc as plsc

NC, NS, L = 2, 16, 16
NW = NC * NS
BATCH, SEQ, HID = 16384, 200, 64
SEQ_PER_W = BATCH // NW
NSEQ_CHUNK = 2
N_ITERS = SEQ_PER_W // NSEQ_CHUNK
IDX_ROW = 100
G = NSEQ_CHUNK * SEQ // IDX_ROW
NBUF = 4
K = 2


def _body(idx_hbm, wtok_hbm, wpos_hbm, out_hbm, idx_v, rows_v, pos_v,
          gsem, osem):
    c = lax.axis_index("c")
    s = lax.axis_index("s")
    wid = s * NC + c
    pltpu.sync_copy(wpos_hbm.at[pl.ds(0, SEQ)], pos_v)
    seq0 = wid * SEQ_PER_W

    def gather_pairs(b):
        for g in range(G):
            yield (idx_v.at[b, g], rows_v.at[b, pl.ds(g * IDX_ROW, IDX_ROW)])

    def fire_gathers(it, b):
        row0 = (seq0 + it * NSEQ_CHUNK) * (SEQ // IDX_ROW)
        pltpu.sync_copy(idx_hbm.at[pl.ds(row0, G)], idx_v.at[b])
        for isrc, rdst in gather_pairs(b):
            pltpu.async_copy(wtok_hbm.at[isrc], rdst, gsem.at[b])

    def wait_gathers(b):
        for isrc, rdst in gather_pairs(b):
            pltpu.make_async_copy(wtok_hbm.at[isrc], rdst, gsem.at[b]).wait()

    def write_pairs(b, it):
        for n in range(NSEQ_CHUNK):
            yield (rows_v.at[b, pl.ds(n * SEQ, SEQ)],
                   out_hbm.at[seq0 + it * NSEQ_CHUNK + n, pl.ds(0, SEQ),
                              pl.ds(0, HID)])

    def wait_write(b, it):
        for src, dst in write_pairs(b, it):
            pltpu.make_async_copy(src, dst, osem.at[b]).wait()

    def add_pos(b):
        @plsc.parallel_loop(0, SEQ, unroll=2)
        def _add_row(j):
            for q in range(HID // L):
                sl = pl.ds(q * L, L)
                p = pos_v[j, sl]
                for n in range(NSEQ_CHUNK):
                    plsc.addupdate(rows_v.at[b, n * SEQ + j, sl], p)

    for b in range(K):
        fire_gathers(b, b)

    def superstep(ss, carry):
        for b in range(NBUF):
            it = ss * NBUF + b
            nxt = it + K
            pb = (b + K) % NBUF
            @pl.when(jnp.logical_and(nxt >= NBUF, nxt < N_ITERS))
            def _():
                wait_write(pb, nxt - NBUF)
            @pl.when(nxt < N_ITERS)
            def _():
                fire_gathers(nxt, pb)
            wait_gathers(b)
            add_pos(b)
            for src_, dst_ in write_pairs(b, it):
                pltpu.async_copy(src_, dst_, osem.at[b])
        return carry

    lax.fori_loop(0, N_ITERS // NBUF, superstep, 0)

    for it in range(N_ITERS - K, N_ITERS):
        wait_write(it % NBUF, it)


_mesh = plsc.VectorSubcoreMesh(core_axis_name="c", subcore_axis_name="s")

_gather_add = pl.kernel(
    _body,
    mesh=_mesh,
    compiler_params=pltpu.CompilerParams(use_tc_tiling_on_sc=False),
    out_type=jax.ShapeDtypeStruct((BATCH, SEQ, 128), jnp.float32),
    scratch_types=[
        pltpu.VMEM((NBUF, G, IDX_ROW), jnp.int32),
        pltpu.VMEM((NBUF, NSEQ_CHUNK * SEQ, HID), jnp.float32),
        pltpu.VMEM((SEQ, HID), jnp.float32),
        pltpu.SemaphoreType.DMA((NBUF,)),
        pltpu.SemaphoreType.DMA((NBUF,)),
    ],
)


def kernel(input_X, W_tok, W_pos):
    idx = input_X.astype(jnp.int32).reshape(BATCH * SEQ // IDX_ROW, IDX_ROW)
    o = _gather_add(idx, W_tok, W_pos)
    return o[:, :, 0:HID]
